# Initial kernel scaffold; baseline (speedup 1.0000x reference)
import jax
import jax.numpy as jnp
from jax.experimental import pallas as pl


def kernel(adj1, adj2, w1, w2, w3, w4, w5, w6, w7, m1_w, m1_b, m2_w, m2_b, m3_w, m3_b):
    raise NotImplementedError("write your pallas kernel here")


# speedup vs baseline: 1.3415x; 1.3415x over previous
import functools

import jax
import jax.numpy as jnp
from jax.experimental import pallas as pl
from jax.experimental.pallas import tpu as pltpu

N = 10000
H = 128
BM0 = 200
BM = 400


def _l2norm(x):
    n = jnp.sqrt(jnp.sum(x * x, axis=1, keepdims=True))
    return x / jnp.maximum(n, 1e-12)


def _mlp(x, m1_w, m1_b, m2_w, m2_b, m3_wr, m3_b):
    h1 = jnp.maximum(
        jnp.dot(x, m1_w, preferred_element_type=jnp.float32) + m1_b, 0.0)
    h2 = jnp.maximum(
        jnp.dot(h1, m2_w, preferred_element_type=jnp.float32) + m2_b, 0.0)
    return jnp.sum(h2 * m3_wr, axis=1, keepdims=True) + m3_b


def _layer0_kernel(adj_ref, w1_ref, wn_ref, m1w_ref, m1b_ref, m2w_ref,
                   m2b_ref, m3wr_ref, m3b_ref,
                   adjbf_ref, hn_ref, s_ref):
    abf = adj_ref[...].astype(jnp.bfloat16)
    adjbf_ref[...] = abf
    acc = jnp.dot(abf, w1_ref[...], preferred_element_type=jnp.float32)
    x = _l2norm(jnp.maximum(acc, 0.0))
    hn_ref[...] = jnp.dot(
        x, wn_ref[...], preferred_element_type=jnp.float32
    ).astype(jnp.bfloat16)
    s_ref[...] = _mlp(x, m1w_ref[...], m1b_ref[...], m2w_ref[...],
                      m2b_ref[...], m3wr_ref[...], m3b_ref[...])


def _layer_kernel(adj_ref, h_ref, wn_ref, m1w_ref, m1b_ref, m2w_ref,
                  m2b_ref, m3wr_ref, m3b_ref,
                  hn_ref, s_ref):
    acc = jnp.dot(adj_ref[...], h_ref[...],
                  preferred_element_type=jnp.float32)
    x = _l2norm(jnp.maximum(acc, 0.0))
    hn_ref[...] = jnp.dot(
        x, wn_ref[...], preferred_element_type=jnp.float32
    ).astype(jnp.bfloat16)
    s_ref[...] = _mlp(x, m1w_ref[...], m1b_ref[...], m2w_ref[...],
                      m2b_ref[...], m3wr_ref[...], m3b_ref[...])


def _last_kernel(adj_ref, h_ref, m1w_ref, m1b_ref, m2w_ref, m2b_ref,
                 m3wr_ref, m3b_ref, s_ref):
    acc = jnp.dot(adj_ref[...], h_ref[...],
                  preferred_element_type=jnp.float32)
    x = jnp.maximum(acc, 0.0)
    s_ref[...] = _mlp(x, m1w_ref[...], m1b_ref[...], m2w_ref[...],
                      m2b_ref[...], m3wr_ref[...], m3b_ref[...])


def _full(shape):
    return pl.BlockSpec(shape, lambda i: (0, 0))


def _branch(adj, w1bf, ws, mlp_args, interpret=False):
    mlp_specs = [_full((H, H)), _full((1, H)), _full((H, H)), _full((1, H)),
                 _full((1, H)), _full((1, 1))]
    params = pltpu.CompilerParams(dimension_semantics=("parallel",))

    adj_bf, h, s = pl.pallas_call(
        _layer0_kernel,
        grid=(N // BM0,),
        in_specs=[pl.BlockSpec((BM0, N), lambda i: (i, 0)),
                  _full((N, H)), _full((H, H))] + mlp_specs,
        out_specs=[pl.BlockSpec((BM0, N), lambda i: (i, 0)),
                   pl.BlockSpec((BM0, H), lambda i: (i, 0)),
                   pl.BlockSpec((BM0, 1), lambda i: (i, 0))],
        out_shape=[jax.ShapeDtypeStruct((N, N), jnp.bfloat16),
                   jax.ShapeDtypeStruct((N, H), jnp.bfloat16),
                   jax.ShapeDtypeStruct((N, 1), jnp.float32)],
        compiler_params=params,
        interpret=interpret,
    )(adj, w1bf, ws[0], *mlp_args)

    for k in range(1, 6):
        h, sk = pl.pallas_call(
            _layer_kernel,
            grid=(N // BM,),
            in_specs=[pl.BlockSpec((BM, N), lambda i: (i, 0)),
                      _full((N, H)), _full((H, H))] + mlp_specs,
            out_specs=[pl.BlockSpec((BM, H), lambda i: (i, 0)),
                       pl.BlockSpec((BM, 1), lambda i: (i, 0))],
            out_shape=[jax.ShapeDtypeStruct((N, H), jnp.bfloat16),
                       jax.ShapeDtypeStruct((N, 1), jnp.float32)],
            compiler_params=params,
            interpret=interpret,
        )(adj_bf, h, ws[k], *mlp_args)
        s = s + sk

    s6 = pl.pallas_call(
        _last_kernel,
        grid=(N // BM,),
        in_specs=[pl.BlockSpec((BM, N), lambda i: (i, 0)),
                  _full((N, H))] + mlp_specs,
        out_specs=pl.BlockSpec((BM, 1), lambda i: (i, 0)),
        out_shape=jax.ShapeDtypeStruct((N, 1), jnp.float32),
        compiler_params=params,
        interpret=interpret,
    )(adj_bf, h, *mlp_args)
    return s + s6


def kernel(adj1, adj2, w1, w2, w3, w4, w5, w6, w7, m1_w, m1_b, m2_w, m2_b,
           m3_w, m3_b, interpret=False):
    ws = [w2, w3, w4, w5, w6, w7]
    mlp_args = (m1_w, m1_b.reshape(1, H), m2_w, m2_b.reshape(1, H),
                m3_w.reshape(1, H), m3_b.reshape(1, 1))
    w1bf = w1.astype(jnp.bfloat16)
    s1 = _branch(adj1, w1bf, ws, mlp_args, interpret=interpret)
    s2 = _branch(adj2, w1bf, ws, mlp_args, interpret=interpret)
    return s1 * s2
